# Initial kernel scaffold; baseline (speedup 1.0000x reference)
import jax
import jax.numpy as jnp
from jax.experimental import pallas as pl


def kernel(pred_images, target_runoff_values, station_rows, station_cols):
    raise NotImplementedError("write your pallas kernel here")


# speedup vs baseline: 1.0932x; 1.0932x over previous
import functools

import jax
import jax.numpy as jnp
from jax import lax
from jax.experimental import pallas as pl
from jax.experimental.pallas import tpu as pltpu
from jax.experimental.pallas import tpu_sc as plsc


def _make_sc_loss(B, HW, W, n_pad, n_valid):
    info = plsc.get_sparse_core_info()
    NC, NS, L = info.num_cores, info.num_subcores, info.num_lanes
    NW = NC * NS
    SPT = n_pad // NW
    E = SPT * B
    NROW = E // 128
    mesh = plsc.VectorSubcoreMesh(core_axis_name="c", subcore_axis_name="s")

    @functools.partial(
        pl.kernel,
        out_type=jax.ShapeDtypeStruct((NW, L), jnp.float32),
        mesh=mesh,
        scratch_types=[
            pltpu.VMEM((SPT,), jnp.int32),
            pltpu.VMEM((SPT,), jnp.int32),
            pltpu.VMEM((SPT,), jnp.int32),
            pltpu.VMEM((NROW, 128), jnp.int32),
            pltpu.VMEM((NROW, 128), jnp.float32),
            pltpu.VMEM((E,), jnp.float32),
            pltpu.VMEM((SPT,), jnp.float32),
            pltpu.VMEM((L,), jnp.float32),
            pltpu.SemaphoreType.DMA,
        ],
    )
    def sc_loss(pred_hbm, tgt_hbm, rows_hbm, cols_hbm, mask_hbm, parts_hbm,
                rows_v, cols_v, sidx_v, fidx_v, g_v, tgt_v, mask_v, part_v,
                sem):
        wid = lax.axis_index("s") * NC + lax.axis_index("c")
        base_s = wid * SPT
        pltpu.sync_copy(rows_hbm.at[pl.ds(base_s, SPT)], rows_v)
        pltpu.sync_copy(cols_hbm.at[pl.ds(base_s, SPT)], cols_v)
        pltpu.sync_copy(mask_hbm.at[pl.ds(base_s, SPT)], mask_v)
        pltpu.sync_copy(tgt_hbm.at[pl.ds(wid * E, E)], tgt_v)

        for c in range(SPT // L):
            r = rows_v[pl.ds(c * L, L)]
            cc = cols_v[pl.ds(c * L, L)]
            sidx_v[pl.ds(c * L, L)] = r * W + cc

        for b in range(B):
            for c in range(SPT // L):
                o = b * SPT + c * L
                fidx_v[o // 128, pl.ds(o % 128, L)] = \
                    sidx_v[pl.ds(c * L, L)] + b * HW

        copies = [
            pltpu.make_async_copy(pred_hbm.at[fidx_v.at[j]], g_v.at[j], sem)
            for j in range(NROW)
        ]
        for cp in copies:
            cp.start()
        for cp in copies:
            cp.wait()

        acc = jnp.zeros((L,), jnp.float32)
        for b in range(B):
            for c in range(SPT // L):
                o = b * SPT + c * L
                g = g_v[o // 128, pl.ds(o % 128, L)]
                t = tgt_v[pl.ds(b * SPT + c * L, L)]
                acc = acc + jnp.abs(g - t) * mask_v[pl.ds(c * L, L)]
        part_v[...] = acc
        pltpu.sync_copy(part_v, parts_hbm.at[wid])

    return sc_loss


def _reduce_body(scale, parts_ref, out_ref):
    out_ref[...] = (jnp.sum(parts_ref[...]) * scale)[None, None]


def kernel(pred_images, target_runoff_values, station_rows, station_cols):
    B, _, H, W = pred_images.shape
    N = station_rows.shape[0]
    HW = H * W
    NW = 32
    SPT = -(-N // NW)
    SPT = -(-SPT // 8) * 8
    n_pad = SPT * NW

    pred_flat = pred_images.reshape(-1)
    rows_p = jnp.pad(station_rows, (0, n_pad - N))
    cols_p = jnp.pad(station_cols, (0, n_pad - N))
    SPT_ = n_pad // NW
    tgt_p = jnp.pad(target_runoff_values[:, :B], ((0, n_pad - N), (0, 0)))
    tgt_prep = tgt_p.reshape(NW, SPT_, B).transpose(0, 2, 1).reshape(-1)
    mask = (jnp.arange(n_pad, dtype=jnp.int32) < N).astype(jnp.float32)

    parts = _make_sc_loss(B, HW, W, n_pad, N)(
        pred_flat, tgt_prep, rows_p, cols_p, mask)

    out = pl.pallas_call(
        functools.partial(_reduce_body, 1.0 / (B * N)),
        out_shape=jax.ShapeDtypeStruct((1, 1), jnp.float32),
    )(parts)
    return out[0, 0]
